# Initial kernel scaffold; baseline (speedup 1.0000x reference)
import jax
import jax.numpy as jnp
from jax.experimental import pallas as pl


def kernel(x, edge_index, W1, b1, W2, b2):
    raise NotImplementedError("write your pallas kernel here")


# speedup vs baseline: 11.9347x; 11.9347x over previous
import functools

import jax
import jax.numpy as jnp
from jax import lax
from jax.experimental import pallas as pl
from jax.experimental.pallas import tpu as pltpu
from jax.experimental.pallas import tpu_sc as plsc

_N = 10000
_E = 320000
_NPAD = 10112
_EPAD = 323584
_CH = 128
_EPT = _EPAD // 32
_NCHUNK = _EPT // _CH
_RPT = _NPAD // 16


def _make_hist():
    mesh = plsc.VectorSubcoreMesh(core_axis_name="c", subcore_axis_name="s")

    @functools.partial(
        pl.kernel,
        out_type=jax.ShapeDtypeStruct((2, _NPAD, 128), jnp.float32),
        mesh=mesh,
        scratch_types=[
            pltpu.VMEM((_CH,), jnp.int32),
            pltpu.VMEM((_CH, 128), jnp.float32),
            pltpu.VMEM_SHARED((_NPAD, 128), jnp.float32),
        ],
    )
    def hist(dst_hbm, ones_hbm, zeros_hbm, out_hbm, dst_v, ones_v, acc):
        c = lax.axis_index("c")
        s = lax.axis_index("s")
        r0 = s * _RPT
        pltpu.sync_copy(zeros_hbm, acc.at[pl.ds(r0, _RPT)])
        pltpu.sync_copy(ones_hbm, ones_v)
        plsc.subcore_barrier()
        base = c * (_EPAD // 2) + s * _EPT

        def body(i, carry):
            off = base + i * _CH
            pltpu.sync_copy(dst_hbm.at[pl.ds(off, _CH)], dst_v)
            pltpu.sync_copy(ones_v, acc.at[dst_v], add=True)
            return carry

        lax.fori_loop(0, _NCHUNK, body, 0)
        plsc.subcore_barrier()
        pltpu.sync_copy(acc.at[pl.ds(r0, _RPT)], out_hbm.at[c, pl.ds(r0, _RPT)])

    return hist


def _make_scatter():
    mesh = plsc.VectorSubcoreMesh(core_axis_name="c", subcore_axis_name="s")

    @functools.partial(
        pl.kernel,
        out_type=jax.ShapeDtypeStruct((2, _NPAD, 128), jnp.float32),
        mesh=mesh,
        scratch_types=[
            pltpu.VMEM((_CH,), jnp.int32),
            pltpu.VMEM((_CH,), jnp.int32),
            pltpu.VMEM((_CH, 128), jnp.float32),
            pltpu.VMEM_SHARED((_NPAD, 128), jnp.float32),
            pltpu.SemaphoreType.DMA,
        ],
    )
    def scat(y_hbm, src_hbm, dst_hbm, zeros_hbm, out_hbm,
             src_v, dst_v, rows_v, acc, sem):
        c = lax.axis_index("c")
        s = lax.axis_index("s")
        r0 = s * _RPT

        @pl.when(c == 0)
        def _():
            pltpu.sync_copy(y_hbm.at[pl.ds(r0, _RPT)], acc.at[pl.ds(r0, _RPT)])

        @pl.when(c != 0)
        def _():
            pltpu.sync_copy(zeros_hbm, acc.at[pl.ds(r0, _RPT)])

        plsc.subcore_barrier()
        base = c * (_EPAD // 2) + s * _EPT

        def body(i, carry):
            off = base + i * _CH
            pltpu.sync_copy(src_hbm.at[pl.ds(off, _CH)], src_v)
            pltpu.sync_copy(dst_hbm.at[pl.ds(off, _CH)], dst_v)
            pltpu.async_copy(y_hbm.at[src_v], rows_v, sem).wait()
            pltpu.sync_copy(rows_v, acc.at[dst_v], add=True)
            return carry

        lax.fori_loop(0, _NCHUNK, body, 0)
        plsc.subcore_barrier()
        pltpu.sync_copy(acc.at[pl.ds(r0, _RPT)], out_hbm.at[c, pl.ds(r0, _RPT)])

    return scat


_hist = _make_hist()
_scatter = _make_scatter()


def _dinv_from(deg_p):
    deg = deg_p[0, :, 0] + deg_p[1, :, 0] + 1.0
    return lax.rsqrt(deg)


def _k1_body(degp_ref, x_ref, xs_ref):
    dinv = _dinv_from(degp_ref[...])
    xs_ref[...] = x_ref[...] * dinv[:, None]


def _k2_body(p_ref, degp_ref, w1_ref, b1_ref, w2_ref, y2s_ref):
    dinv = _dinv_from(degp_ref[...])
    z1 = (p_ref[0] + p_ref[1]) * dinv[:, None]
    h = jnp.dot(z1, w1_ref[...], preferred_element_type=jnp.float32)
    h = jnp.maximum(h + b1_ref[...], 0.0)
    y2 = jnp.dot(h, w2_ref[...], preferred_element_type=jnp.float32)
    y2s_ref[...] = y2 * dinv[:, None]


def _k3_body(p_ref, degp_ref, b2_ref, out_ref):
    dinv = _dinv_from(degp_ref[...])
    z = (p_ref[0] + p_ref[1]) * dinv[:, None] + b2_ref[...]
    out_ref[...] = z[:_N]


@jax.jit
def kernel(x, edge_index, W1, b1, W2, b2):
    src = edge_index[0].astype(jnp.int32)
    dst = edge_index[1].astype(jnp.int32)
    pad = _EPAD - _E
    src_p = jnp.concatenate([src, jnp.zeros((pad,), jnp.int32)])
    dst_p = jnp.concatenate([dst, jnp.full((pad,), _N, jnp.int32)])
    x_p = jnp.pad(x, ((0, _NPAD - _N), (0, 0)))
    ones128 = jnp.ones((_CH, 128), jnp.float32)
    zeros128 = jnp.zeros((_RPT, 128), jnp.float32)

    deg_p = _hist(dst_p, ones128, zeros128)

    xs = pl.pallas_call(
        _k1_body,
        out_shape=jax.ShapeDtypeStruct((_NPAD, 128), jnp.float32),
    )(deg_p, x_p)

    p1 = _scatter(xs, src_p, dst_p, zeros128)

    y2s = pl.pallas_call(
        _k2_body,
        out_shape=jax.ShapeDtypeStruct((_NPAD, 128), jnp.float32),
    )(p1, deg_p, W1, b1.reshape(1, -1), W2)

    p2 = _scatter(y2s, src_p, dst_p, zeros128)

    out = pl.pallas_call(
        _k3_body,
        out_shape=jax.ShapeDtypeStruct((_N, 128), jnp.float32),
    )(p2, deg_p, b2.reshape(1, -1))

    return out
